# baseline (speedup 1.0000x reference)
import functools

import jax
import jax.numpy as jnp
from jax import lax
from jax.experimental import pallas as pl
from jax.experimental.pallas import tpu as pltpu
from jax.experimental.pallas import tpu_sc as plsc

_N_WORKERS = 32
_E_ROWS = 320000
_ROWS_PER_W = _E_ROWS // _N_WORKERS
_ECHUNK = 2000
_NCHUNK = _ROWS_PER_W // _ECHUNK
_SLOTS = 3


@functools.partial(
    pl.kernel,
    out_type=jax.ShapeDtypeStruct((_E_ROWS, 16), jnp.float32),
    mesh=plsc.VectorSubcoreMesh(core_axis_name="c", subcore_axis_name="s"),
    compiler_params=pltpu.CompilerParams(use_tc_tiling_on_sc=False),
    scratch_types=[
        pltpu.VMEM((_SLOTS, _ECHUNK, 16), jnp.float32),
        pltpu.SemaphoreType.DMA((_SLOTS,)),
        pltpu.SemaphoreType.DMA((_SLOTS,)),
    ],
)
def _sc_copy_edge(e_hbm, out_hbm, buf, in_sems, out_sems):
    wid = lax.axis_index("s") * 2 + lax.axis_index("c")
    base = pl.multiple_of(wid * _ROWS_PER_W, 8)

    def _in_copy(j):
        b = jnp.int32(j % _SLOTS)
        row = pl.multiple_of(base + jnp.int32(j * _ECHUNK), 8)
        return pltpu.make_async_copy(
            e_hbm.at[pl.ds(row, _ECHUNK), :], buf.at[b], in_sems.at[b])

    def _out_copy(j):
        b = jnp.int32(j % _SLOTS)
        row = pl.multiple_of(base + jnp.int32(j * _ECHUNK), 8)
        return pltpu.make_async_copy(
            buf.at[b], out_hbm.at[pl.ds(row, _ECHUNK), :], out_sems.at[b])

    for j in range(_SLOTS):
        _in_copy(j).start()
    out_copies = {}
    for j in range(_NCHUNK):
        _in_copy(j).wait()
        co = _out_copy(j)
        co.start()
        out_copies[j] = co
        nxt = j + _SLOTS
        if nxt < _NCHUNK:
            out_copies.pop(j).wait()
            _in_copy(nxt).start()
    for j in list(out_copies):
        out_copies.pop(j).wait()


_XGRID = 10
_XB = 10000 // _XGRID


def _tc_copy_body(x_ref, xo_ref):
    xo_ref[...] = x_ref[...]


def _tc_copy_x(x):
    return pl.pallas_call(
        _tc_copy_body,
        grid=(_XGRID,),
        in_specs=[pl.BlockSpec((_XB, 128), lambda i: (i, jnp.int32(0)))],
        out_specs=pl.BlockSpec((_XB, 128), lambda i: (i, jnp.int32(0))),
        out_shape=jax.ShapeDtypeStruct((10000, 128), jnp.float32),
        compiler_params=pltpu.CompilerParams(
            dimension_semantics=("arbitrary",),
        ),
    )(x)


def kernel(x, edge_index, edge_attr):
    del edge_index
    e_out = _sc_copy_edge(edge_attr)
    x_out = _tc_copy_x(x)
    return (x_out, e_out)


# speedup vs baseline: 1.1056x; 1.1056x over previous
import jax
import jax.numpy as jnp
from jax.experimental import pallas as pl
from jax.experimental.pallas import tpu as pltpu

_GRID = 25
_XB = 10000 // _GRID
_EB = 320000 // _GRID


def _copy_body(x_ref, e_ref, xo_ref, eo_ref):
    xo_ref[...] = x_ref[...]
    eo_ref[...] = e_ref[...]


def kernel(x, edge_index, edge_attr):
    del edge_index
    x_out, e_out = pl.pallas_call(
        _copy_body,
        grid=(_GRID,),
        in_specs=[
            pl.BlockSpec((_XB, 128), lambda i: (i, jnp.int32(0))),
            pl.BlockSpec((_EB, 16), lambda i: (i, jnp.int32(0))),
        ],
        out_specs=[
            pl.BlockSpec((_XB, 128), lambda i: (i, jnp.int32(0))),
            pl.BlockSpec((_EB, 16), lambda i: (i, jnp.int32(0))),
        ],
        out_shape=[
            jax.ShapeDtypeStruct(x.shape, x.dtype),
            jax.ShapeDtypeStruct(edge_attr.shape, edge_attr.dtype),
        ],
        compiler_params=pltpu.CompilerParams(
            dimension_semantics=("arbitrary",),
        ),
    )(x, edge_attr)
    return (x_out, e_out)
